# baseline (speedup 1.0000x reference)
import functools

import jax
import jax.numpy as jnp
from jax import lax
from jax.experimental import pallas as pl
from jax.experimental.pallas import tpu as pltpu
from jax.experimental.pallas import tpu_sc as plsc

GRID = 129
G2 = GRID * GRID
G3 = GRID * GRID * GRID
NC = 2
NS = 16
NW = NC * NS
CH = 8192
LANES = 16
NBUF = 2


def _vgather(v, perm):
    return lax.gather(
        v,
        perm[:, None],
        lax.GatherDimensionNumbers(
            offset_dims=(), collapsed_slice_dims=(0,), start_index_map=(0,)),
        (1,),
        mode=lax.GatherScatterMode.PROMISE_IN_BOUNDS,
    )


def _body(asu_hbm, hkl_hbm, tab_hbm, out_hbm,
          asu_v, hkl_v, idx_v, o_v, sin, sg, sout):
    wid = lax.axis_index("s") * NC + lax.axis_index("c")
    b_per_w = asu_hbm.shape[0] // NW
    n_chunks = b_per_w // CH
    base = wid * b_per_w

    lane = lax.iota(jnp.int32, LANES)
    w_c = (G2, GRID, 1)
    wpat = []
    for m in range(3):
        q = LANES * m + lane
        ph = q % 3
        wpat.append(jnp.where(ph == 0, G2, jnp.where(ph == 1, GRID, 1)))
    perms = []
    masks = []
    for c in range(3):
        q = 3 * lane + c
        perms.append(q & (LANES - 1))
        mm = q >> 4
        masks.append((mm == 0, mm == 1))

    def start_in(c):
        p = c % NBUF
        off = base + c * CH
        pltpu.make_async_copy(asu_hbm.at[pl.ds(off, CH)], asu_v.at[p], sin[p]).start()
        pltpu.make_async_copy(hkl_hbm.at[pl.ds(off * 3, CH * 3)], hkl_v.at[p], sin[p]).start()

    def wait_in(c):
        p = c % NBUF
        pltpu.make_async_copy(asu_hbm.at[pl.ds(base, CH)], asu_v.at[p], sin[p]).wait()
        pltpu.make_async_copy(hkl_hbm.at[pl.ds(base, CH * 3)], hkl_v.at[p], sin[p]).wait()

    def compute_idx(c):
        p = c % NBUF

        def vec_body(i, carry):
            q0 = i * (3 * LANES)
            v = [hkl_v[p, pl.ds(q0 + LANES * m, LANES)] * wpat[m] for m in range(3)]
            acc = asu_v[p, pl.ds(i * LANES, LANES)] * G3
            for cc in range(3):
                g0 = _vgather(v[0], perms[cc])
                g1 = _vgather(v[1], perms[cc])
                g2 = _vgather(v[2], perms[cc])
                comp = jnp.where(masks[cc][0], g0, jnp.where(masks[cc][1], g1, g2))
                acc = acc + comp
            idx_v[p][pl.ds(i * LANES, LANES)] = acc
            return carry

        lax.fori_loop(0, CH // LANES, vec_body, 0, unroll=4)

    def start_gather(c):
        p = c % NBUF
        pltpu.make_async_copy(tab_hbm.at[idx_v[p]], o_v[c], sg[p]).start()

    def wait_gather(c):
        p = c % NBUF
        pltpu.make_async_copy(tab_hbm.at[idx_v[p]], o_v[c], sg[p]).wait()

    def start_out(c):
        off = base + c * CH
        pltpu.make_async_copy(o_v[c], out_hbm.at[pl.ds(off, CH)], sout).start()

    def wait_out(c):
        off = base + c * CH
        pltpu.make_async_copy(o_v[c], out_hbm.at[pl.ds(off, CH)], sout).wait()

    start_in(0)
    start_in(1)
    for c in range(n_chunks):
        if c >= NBUF:
            wait_gather(c - NBUF)
            start_out(c - NBUF)
        wait_in(c)
        compute_idx(c)
        start_gather(c)
        if c + NBUF < n_chunks:
            start_in(c + NBUF)
    for c in range(n_chunks - NBUF, n_chunks):
        wait_gather(c)
        start_out(c)
    for c in range(n_chunks):
        wait_out(c)


def kernel(asu_id, hkl, miller_id):
    B = asu_id.shape[0]
    asu32 = asu_id.astype(jnp.int32)
    hklp = hkl.astype(jnp.int32).reshape(-1)
    tab = miller_id.reshape(-1)
    n_chunks = B // NW // CH

    mesh = plsc.VectorSubcoreMesh(core_axis_name="c", subcore_axis_name="s")
    run = functools.partial(
        pl.kernel,
        mesh=mesh,
        out_type=jax.ShapeDtypeStruct((B,), jnp.int32),
        scratch_types=[
            pltpu.VMEM((NBUF, CH), jnp.int32),
            pltpu.VMEM((NBUF, CH * 3), jnp.int32),
            [pltpu.VMEM((CH,), jnp.int32)] * NBUF,
            [pltpu.VMEM((CH,), jnp.int32)] * n_chunks,
            [pltpu.SemaphoreType.DMA] * NBUF,
            [pltpu.SemaphoreType.DMA] * NBUF,
            pltpu.SemaphoreType.DMA,
        ],
    )(_body)
    return run(asu32, hklp, tab)


# speedup vs baseline: 1.0267x; 1.0267x over previous
import functools

import numpy as np
import jax
import jax.numpy as jnp
from jax import lax
from jax.experimental import pallas as pl
from jax.experimental.pallas import tpu as pltpu
from jax.experimental.pallas import tpu_sc as plsc

GRID = 129
G2 = GRID * GRID
G3 = GRID * GRID * GRID
NC = 2
NS = 16
NW = NC * NS
CH = 16384
LANES = 128
RB = 512

_W = np.zeros((3 * LANES, LANES), np.float32)
for _t in range(LANES):
    _W[3 * _t + 0, _t] = G2
    _W[3 * _t + 1, _t] = GRID
    _W[3 * _t + 2, _t] = 1.0


def _idx_body(asu_ref, hkl_ref, w_ref, idx_ref):
    y = jnp.dot(hkl_ref[...].astype(jnp.float32), w_ref[...],
                preferred_element_type=jnp.float32,
                precision=lax.Precision.HIGHEST)
    idx_ref[...] = asu_ref[...] * G3 + y.astype(jnp.int32)


def _flat_indices(asu32, hklp):
    B = asu32.shape[0]
    nr = B // LANES
    grid = nr // RB
    return pl.pallas_call(
        _idx_body,
        grid=(grid,),
        in_specs=[
            pl.BlockSpec((RB, LANES), lambda i: (i, 0)),
            pl.BlockSpec((RB, 3 * LANES), lambda i: (i, 0)),
            pl.BlockSpec((3 * LANES, LANES), lambda i: (0, 0)),
        ],
        out_specs=pl.BlockSpec((RB, LANES), lambda i: (i, 0)),
        out_shape=jax.ShapeDtypeStruct((nr, LANES), jnp.int32),
    )(asu32.reshape(nr, LANES), hklp.reshape(nr, 3 * LANES), jnp.asarray(_W))


def _gather_body(idx_hbm, tab_hbm, out_hbm,
                 idx_v, o_v, sin, sg, sout):
    wid = lax.axis_index("s") * NC + lax.axis_index("c")
    b_per_w = idx_hbm.shape[0] // NW
    n_chunks = b_per_w // CH
    base = wid * b_per_w

    def start_in(c):
        off = base + c * CH
        pltpu.make_async_copy(idx_hbm.at[pl.ds(off, CH)], idx_v[c], sin[c]).start()

    def wait_in(c):
        off = base + c * CH
        pltpu.make_async_copy(idx_hbm.at[pl.ds(off, CH)], idx_v[c], sin[c]).wait()

    def start_gather(c):
        pltpu.make_async_copy(tab_hbm.at[idx_v[c]], o_v[c], sg[c]).start()

    def wait_gather(c):
        pltpu.make_async_copy(tab_hbm.at[idx_v[c]], o_v[c], sg[c]).wait()

    def start_out(c):
        off = base + c * CH
        pltpu.make_async_copy(o_v[c], out_hbm.at[pl.ds(off, CH)], sout).start()

    def wait_out(c):
        off = base + c * CH
        pltpu.make_async_copy(o_v[c], out_hbm.at[pl.ds(off, CH)], sout).wait()

    for c in range(n_chunks):
        start_in(c)
    for c in range(n_chunks):
        wait_in(c)
        start_gather(c)
    for c in range(n_chunks):
        wait_gather(c)
        start_out(c)
    for c in range(n_chunks):
        wait_out(c)


def kernel(asu_id, hkl, miller_id):
    B = asu_id.shape[0]
    asu32 = asu_id.astype(jnp.int32)
    hklp = hkl.astype(jnp.int32).reshape(-1)
    tab = miller_id.reshape(-1)
    idx = _flat_indices(asu32, hklp).reshape(-1)
    n_chunks = B // NW // CH

    mesh = plsc.VectorSubcoreMesh(core_axis_name="c", subcore_axis_name="s")
    run = functools.partial(
        pl.kernel,
        mesh=mesh,
        out_type=jax.ShapeDtypeStruct((B,), jnp.int32),
        scratch_types=[
            [pltpu.VMEM((CH,), jnp.int32)] * n_chunks,
            [pltpu.VMEM((CH,), jnp.int32)] * n_chunks,
            [pltpu.SemaphoreType.DMA] * n_chunks,
            [pltpu.SemaphoreType.DMA] * n_chunks,
            pltpu.SemaphoreType.DMA,
        ],
    )(_gather_body)
    return run(idx, tab)


# speedup vs baseline: 5.0428x; 4.9116x over previous
import functools

import jax
import jax.numpy as jnp
from jax import lax
from jax.experimental import pallas as pl
from jax.experimental.pallas import tpu as pltpu
from jax.experimental.pallas import tpu_sc as plsc

GRID = 129
G2 = GRID * GRID
G3 = GRID * GRID * GRID
NC = 2
NS = 16
NW = NC * NS
CH = 16384
LANES = 16


def _gather_body(hidx_hbm, asu_hbm, tab_hbm, out_hbm,
                 h_v, a_v, o_v, sin, sg, sout):
    wid = lax.axis_index("s") * NC + lax.axis_index("c")
    b_per_w = hidx_hbm.shape[0] // NW
    n_chunks = b_per_w // CH
    base = wid * b_per_w

    def start_in(c):
        off = base + c * CH
        pltpu.make_async_copy(hidx_hbm.at[pl.ds(off, CH)], h_v[c], sin[c]).start()
        pltpu.make_async_copy(asu_hbm.at[pl.ds(off, CH)], a_v[c], sin[c]).start()

    def wait_in(c):
        off = base + c * CH
        pltpu.make_async_copy(hidx_hbm.at[pl.ds(off, CH)], h_v[c], sin[c]).wait()
        pltpu.make_async_copy(asu_hbm.at[pl.ds(off, CH)], a_v[c], sin[c]).wait()

    def compute_idx(c):
        def vec_body(i, carry):
            s = pl.ds(i * LANES, LANES)
            h_v[c][s] = h_v[c][s] + a_v[c][s] * G3
            return carry

        lax.fori_loop(0, CH // LANES, vec_body, 0, unroll=8)

    def start_gather(c):
        pltpu.make_async_copy(tab_hbm.at[h_v[c]], o_v[c], sg[c]).start()

    def wait_gather(c):
        pltpu.make_async_copy(tab_hbm.at[h_v[c]], o_v[c], sg[c]).wait()

    def start_out(c):
        off = base + c * CH
        pltpu.make_async_copy(o_v[c], out_hbm.at[pl.ds(off, CH)], sout).start()

    def wait_out(c):
        off = base + c * CH
        pltpu.make_async_copy(o_v[c], out_hbm.at[pl.ds(off, CH)], sout).wait()

    for c in range(n_chunks):
        start_in(c)
    for c in range(n_chunks):
        wait_in(c)
        compute_idx(c)
        start_gather(c)
    for c in range(n_chunks):
        wait_gather(c)
        start_out(c)
    for c in range(n_chunks):
        wait_out(c)


def kernel(asu_id, hkl, miller_id):
    B = asu_id.shape[0]
    asu32 = asu_id.astype(jnp.int32)
    hkl32 = hkl.astype(jnp.int32)
    hklT = jnp.swapaxes(hkl32, 0, 1)
    hidx = hklT[0] * G2 + hklT[1] * GRID + hklT[2]
    tab = miller_id.reshape(-1)
    n_chunks = B // NW // CH

    mesh = plsc.VectorSubcoreMesh(core_axis_name="c", subcore_axis_name="s")
    run = functools.partial(
        pl.kernel,
        mesh=mesh,
        out_type=jax.ShapeDtypeStruct((B,), jnp.int32),
        scratch_types=[
            [pltpu.VMEM((CH,), jnp.int32)] * n_chunks,
            [pltpu.VMEM((CH,), jnp.int32)] * n_chunks,
            [pltpu.VMEM((CH,), jnp.int32)] * n_chunks,
            [pltpu.SemaphoreType.DMA] * n_chunks,
            [pltpu.SemaphoreType.DMA] * n_chunks,
            pltpu.SemaphoreType.DMA,
        ],
    )(_gather_body)
    return run(hidx, asu32, tab)


# speedup vs baseline: 6.6248x; 1.3137x over previous
import functools

import jax
import jax.numpy as jnp
from jax import lax
from jax.experimental import pallas as pl
from jax.experimental.pallas import tpu as pltpu
from jax.experimental.pallas import tpu_sc as plsc

GRID = 129
MAIN = GRID * GRID * 2 * 128
NC = 2
NS = 16
NW = NC * NS
CH = 8192
LANES = 16
BLK = 128
NBUF = 2


def _gather_body(pk_hbm, tab_hbm, out_hbm, pk_v, idx_v, o_v, sin, sg, sout):
    wid = lax.axis_index("s") * NC + lax.axis_index("c")
    b_per_w = out_hbm.shape[0] // NW
    n_chunks = b_per_w // CH
    base = wid * b_per_w

    def start_in(c):
        p = c % NBUF
        off = (base + c * CH) * 4
        pltpu.make_async_copy(pk_hbm.at[pl.ds(off, CH * 4)], pk_v[p], sin[p]).start()

    def wait_in(c):
        p = c % NBUF
        off = (base + c * CH) * 4
        pltpu.make_async_copy(pk_hbm.at[pl.ds(off, CH * 4)], pk_v[p], sin[p]).wait()

    def compute_idx(c):
        p = c % NBUF

        def blk_body(j, carry):
            src = j * (4 * BLK)
            dst = j * BLK
            for m in range(BLK // LANES):
                h = pk_v[p][pl.ds(src + m * LANES, LANES)]
                k = pk_v[p][pl.ds(src + BLK + m * LANES, LANES)]
                l = pk_v[p][pl.ds(src + 2 * BLK + m * LANES, LANES)]
                a = pk_v[p][pl.ds(src + 3 * BLK + m * LANES, LANES)]
                pre = (h * GRID + k) * 2 + a
                idx_v[p][pl.ds(dst + m * LANES, LANES)] = jnp.where(
                    l < 128, pre * 128 + l, MAIN + pre
                )
            return carry

        lax.fori_loop(0, CH // BLK, blk_body, 0)

    def start_gather(c):
        p = c % NBUF
        pltpu.make_async_copy(tab_hbm.at[idx_v[p]], o_v[c], sg[p]).start()

    def wait_gather(c):
        p = c % NBUF
        pltpu.make_async_copy(tab_hbm.at[idx_v[p]], o_v[c], sg[p]).wait()

    def start_out(c):
        off = base + c * CH
        pltpu.make_async_copy(o_v[c], out_hbm.at[pl.ds(off, CH)], sout).start()

    def wait_out(c):
        off = base + c * CH
        pltpu.make_async_copy(o_v[c], out_hbm.at[pl.ds(off, CH)], sout).wait()

    for c in range(min(NBUF, n_chunks)):
        start_in(c)
    for c in range(n_chunks):
        if c >= NBUF:
            wait_gather(c - NBUF)
            start_out(c - NBUF)
        wait_in(c)
        compute_idx(c)
        start_gather(c)
        if c + NBUF < n_chunks:
            start_in(c + NBUF)
    for c in range(max(n_chunks - NBUF, 0), n_chunks):
        wait_gather(c)
        start_out(c)
    for c in range(n_chunks):
        wait_out(c)


def kernel(asu_id, hkl, miller_id):
    B = asu_id.shape[0]
    asu32 = asu_id.astype(jnp.int32)
    hkl32 = hkl.astype(jnp.int32)
    pk4 = jnp.concatenate([hkl32, asu32[:, None]], axis=1)
    pk = (
        pk4.T.reshape(4, B // BLK, BLK).transpose(1, 0, 2).reshape(-1)
    )
    main = miller_id[:, :, :, :128].transpose(1, 2, 0, 3).reshape(-1)
    rest = miller_id[:, :, :, 128].transpose(1, 2, 0).reshape(-1)
    tab = jnp.concatenate([main, rest])
    n_chunks = B // NW // CH

    mesh = plsc.VectorSubcoreMesh(core_axis_name="c", subcore_axis_name="s")
    run = functools.partial(
        pl.kernel,
        mesh=mesh,
        out_type=jax.ShapeDtypeStruct((B,), jnp.int32),
        scratch_types=[
            [pltpu.VMEM((CH * 4,), jnp.int32)] * NBUF,
            [pltpu.VMEM((CH,), jnp.int32)] * NBUF,
            [pltpu.VMEM((CH,), jnp.int32)] * n_chunks,
            [pltpu.SemaphoreType.DMA] * NBUF,
            [pltpu.SemaphoreType.DMA] * NBUF,
            pltpu.SemaphoreType.DMA,
        ],
    )(_gather_body)
    return run(pk, tab)


# speedup vs baseline: 10.7601x; 1.6242x over previous
import functools

import jax
import jax.numpy as jnp
from jax import lax
from jax.experimental import pallas as pl
from jax.experimental.pallas import tpu as pltpu
from jax.experimental.pallas import tpu_sc as plsc

GRID = 129
SA = 129 * 17 * 2 * 8 * 128
SH = 17 * 2 * 8 * 128
SKT = 2 * 8 * 128
SLT = 8 * 128
NC = 2
NS = 16
NW = NC * NS
CH = 8192
LANES = 16
BLK = 128
NBUF = 2


def _gather_body(pk_hbm, tab_hbm, out_hbm, pk_v, idx_v, o_v, sin, sg, sout):
    wid = lax.axis_index("s") * NC + lax.axis_index("c")
    b_per_w = out_hbm.shape[0] // NW
    n_chunks = b_per_w // CH
    base = wid * b_per_w

    def start_in(c):
        p = c % NBUF
        off = (base + c * CH) * 4
        pltpu.make_async_copy(pk_hbm.at[pl.ds(off, CH * 4)], pk_v[p], sin[p]).start()

    def wait_in(c):
        p = c % NBUF
        off = (base + c * CH) * 4
        pltpu.make_async_copy(pk_hbm.at[pl.ds(off, CH * 4)], pk_v[p], sin[p]).wait()

    def compute_idx(c):
        p = c % NBUF

        def blk_body(j, carry):
            src = j * (4 * BLK)
            dst = j * BLK
            for m in range(BLK // LANES):
                h = pk_v[p][pl.ds(src + m * LANES, LANES)]
                k = pk_v[p][pl.ds(src + BLK + m * LANES, LANES)]
                l = pk_v[p][pl.ds(src + 2 * BLK + m * LANES, LANES)]
                a = pk_v[p][pl.ds(src + 3 * BLK + m * LANES, LANES)]
                idx_v[p][pl.ds(dst + m * LANES, LANES)] = (
                    a * SA + h * SH
                    + (k >> 3) * SKT + (l >> 7) * SLT
                    + (k & 7) * 128 + (l & 127)
                )
            return carry

        lax.fori_loop(0, CH // BLK, blk_body, 0)

    def start_gather(c):
        p = c % NBUF
        pltpu.make_async_copy(tab_hbm.at[idx_v[p]], o_v[c], sg[p]).start()

    def wait_gather(c):
        p = c % NBUF
        pltpu.make_async_copy(tab_hbm.at[idx_v[p]], o_v[c], sg[p]).wait()

    def start_out(c):
        off = base + c * CH
        pltpu.make_async_copy(o_v[c], out_hbm.at[pl.ds(off, CH)], sout).start()

    def wait_out(c):
        off = base + c * CH
        pltpu.make_async_copy(o_v[c], out_hbm.at[pl.ds(off, CH)], sout).wait()

    for c in range(min(NBUF, n_chunks)):
        start_in(c)
    for c in range(n_chunks):
        if c >= NBUF:
            wait_gather(c - NBUF)
            start_out(c - NBUF)
        wait_in(c)
        compute_idx(c)
        start_gather(c)
        if c + NBUF < n_chunks:
            start_in(c + NBUF)
    for c in range(max(n_chunks - NBUF, 0), n_chunks):
        wait_gather(c)
        start_out(c)
    for c in range(n_chunks):
        wait_out(c)


def kernel(asu_id, hkl, miller_id):
    B = asu_id.shape[0]
    asu32 = asu_id.astype(jnp.int32)
    hkl32 = hkl.astype(jnp.int32)
    pk4 = jnp.concatenate([hkl32, asu32[:, None]], axis=1)
    pk = (
        pk4.T.reshape(4, B // BLK, BLK).transpose(1, 0, 2).reshape(-1)
    )
    padded = jnp.pad(miller_id, ((0, 0), (0, 0), (0, 7), (0, 127)))
    tab = (
        padded.reshape(2, GRID, 17, 8, 2, 128)
        .transpose(0, 1, 2, 4, 3, 5)
        .reshape(-1)
    )
    n_chunks = B // NW // CH

    mesh = plsc.VectorSubcoreMesh(core_axis_name="c", subcore_axis_name="s")
    run = functools.partial(
        pl.kernel,
        mesh=mesh,
        out_type=jax.ShapeDtypeStruct((B,), jnp.int32),
        scratch_types=[
            [pltpu.VMEM((CH * 4,), jnp.int32)] * NBUF,
            [pltpu.VMEM((CH,), jnp.int32)] * NBUF,
            [pltpu.VMEM((CH,), jnp.int32)] * n_chunks,
            [pltpu.SemaphoreType.DMA] * NBUF,
            [pltpu.SemaphoreType.DMA] * NBUF,
            pltpu.SemaphoreType.DMA,
        ],
    )(_gather_body)
    return run(pk, tab)


# speedup vs baseline: 10.8598x; 1.0093x over previous
import functools

import jax
import jax.numpy as jnp
from jax import lax
from jax.experimental import pallas as pl
from jax.experimental.pallas import tpu as pltpu
from jax.experimental.pallas import tpu_sc as plsc

GRID = 129
SA = 129 * 17 * 2 * 8 * 128
SH = 17 * 2 * 8 * 128
SKT = 2 * 8 * 128
SLT = 8 * 128
NC = 2
NS = 16
NW = NC * NS
CH = 8192
LANES = 16
BLK = 128
NBUF = 2


def _gather_body(pk_hbm, tab_hbm, out_hbm, pk_v, idx_v, o_v, sin, sg, sout):
    wid = lax.axis_index("s") * NC + lax.axis_index("c")
    b_per_w = out_hbm.shape[0] // NW
    n_chunks = b_per_w // CH
    base = wid * b_per_w

    def start_in(c):
        p = c % NBUF
        off = base + c * CH
        pltpu.make_async_copy(pk_hbm.at[pl.ds(off, CH)], pk_v[p], sin[p]).start()

    def wait_in(c):
        p = c % NBUF
        off = base + c * CH
        pltpu.make_async_copy(pk_hbm.at[pl.ds(off, CH)], pk_v[p], sin[p]).wait()

    def compute_idx(c):
        p = c % NBUF

        def vec_body(i, carry):
            s = pl.ds(i * LANES, LANES)
            w = pk_v[p][s]
            h = w & 255
            k = (w >> 8) & 255
            l = (w >> 16) & 255
            a = w >> 24
            idx_v[p][s] = (
                a * SA + h * SH
                + (k >> 3) * SKT + (l >> 7) * SLT
                + (k & 7) * 128 + (l & 127)
            )
            return carry

        lax.fori_loop(0, CH // LANES, vec_body, 0, unroll=8)

    def start_gather(c):
        p = c % NBUF
        pltpu.make_async_copy(tab_hbm.at[idx_v[p]], o_v[c], sg[p]).start()

    def wait_gather(c):
        p = c % NBUF
        pltpu.make_async_copy(tab_hbm.at[idx_v[p]], o_v[c], sg[p]).wait()

    def start_out(c):
        off = base + c * CH
        pltpu.make_async_copy(o_v[c], out_hbm.at[pl.ds(off, CH)], sout).start()

    def wait_out(c):
        off = base + c * CH
        pltpu.make_async_copy(o_v[c], out_hbm.at[pl.ds(off, CH)], sout).wait()

    for c in range(min(NBUF, n_chunks)):
        start_in(c)
    for c in range(n_chunks):
        if c >= NBUF:
            wait_gather(c - NBUF)
            start_out(c - NBUF)
        wait_in(c)
        compute_idx(c)
        start_gather(c)
        if c + NBUF < n_chunks:
            start_in(c + NBUF)
    for c in range(max(n_chunks - NBUF, 0), n_chunks):
        wait_gather(c)
        start_out(c)
    for c in range(n_chunks):
        wait_out(c)


def kernel(asu_id, hkl, miller_id):
    B = asu_id.shape[0]
    asu32 = asu_id.astype(jnp.int32)
    hkl32 = hkl.astype(jnp.int32)
    pk = (
        hkl32[:, 0]
        + hkl32[:, 1] * 256
        + hkl32[:, 2] * 65536
        + asu32 * 16777216
    )
    padded = jnp.pad(miller_id, ((0, 0), (0, 0), (0, 7), (0, 127)))
    tab = (
        padded.reshape(2, GRID, 17, 8, 2, 128)
        .transpose(0, 1, 2, 4, 3, 5)
        .reshape(-1)
    )
    n_chunks = B // NW // CH

    mesh = plsc.VectorSubcoreMesh(core_axis_name="c", subcore_axis_name="s")
    run = functools.partial(
        pl.kernel,
        mesh=mesh,
        out_type=jax.ShapeDtypeStruct((B,), jnp.int32),
        scratch_types=[
            [pltpu.VMEM((CH,), jnp.int32)] * NBUF,
            [pltpu.VMEM((CH,), jnp.int32)] * NBUF,
            [pltpu.VMEM((CH,), jnp.int32)] * n_chunks,
            [pltpu.SemaphoreType.DMA] * NBUF,
            [pltpu.SemaphoreType.DMA] * NBUF,
            pltpu.SemaphoreType.DMA,
        ],
    )(_gather_body)
    return run(pk, tab)
